# baseline (speedup 1.0000x reference)
import jax
import jax.numpy as jnp
from jax import lax
from jax.experimental import pallas as pl
from jax.experimental.pallas import tpu as pltpu
from jax.experimental.pallas import tpu_sc as plsc

_NC = 2
_NS = 16
_NW = _NC * _NS
_LANES = 16
_CHUNK = 128


def kernel(idx, W):
    B = idx.shape[0]
    NZ = W.shape[1]
    pack = 128 // NZ
    Wwide = W.reshape(W.shape[0] // pack, 128)
    b_per_w = B // _NW
    n_chunks = b_per_w // _CHUNK

    mesh = plsc.VectorSubcoreMesh(core_axis_name="c", subcore_axis_name="s")

    @pl.kernel(
        mesh=mesh,
        out_type=jax.ShapeDtypeStruct((B, NZ), W.dtype),
        compiler_params=pltpu.CompilerParams(needs_layout_passes=False),
        scratch_types=[
            pltpu.VMEM((b_per_w,), jnp.int32),
            pltpu.VMEM((b_per_w,), jnp.int32),
            pltpu.VMEM((b_per_w, 128), jnp.float32),
            pltpu.VMEM((2, _CHUNK, NZ), jnp.float32),
            pltpu.SemaphoreType.DMA,
            pltpu.SemaphoreType.DMA,
            pltpu.SemaphoreType.DMA,
        ],
    )
    def k(idx_hbm, table_hbm, out_hbm, idx_v, idxq_v, wide_v, out_v, gsem,
          osem0, osem1):
        wid = lax.axis_index("s") * _NC + lax.axis_index("c")
        base = wid * b_per_w
        pltpu.sync_copy(idx_hbm.at[pl.ds(base, b_per_w)], idx_v)

        @pl.loop(0, b_per_w, step=_LANES)
        def _(i):
            idxq_v[pl.ds(i, _LANES)] = idx_v[pl.ds(i, _LANES)] // pack

        gathers = [
            pltpu.async_copy(
                table_hbm.at[idxq_v.at[pl.ds(j * _CHUNK, _CHUNK)]],
                wide_v.at[pl.ds(j * _CHUNK, _CHUNK)],
                gsem,
            )
            for j in range(n_chunks)
        ]
        for g in gathers:
            g.wait()

        lane = lax.iota(jnp.int32, _LANES)
        out_copies = []
        for j in range(n_chunks):
            buf = out_v.at[j % 2]
            if j >= 2:
                out_copies[j - 2].wait()

            @pl.loop(0, _CHUNK, step=_LANES)
            def _(i, j=j, buf=buf):
                rows = lane + (j * _CHUNK + i)
                offs = (idx_v[pl.ds(j * _CHUNK + i, _LANES)] % pack) * NZ
                local = lane + i
                for c in range(NZ):
                    vals = plsc.load_gather(wide_v, [rows, offs + c])
                    plsc.store_scatter(
                        buf, [local, jnp.full((_LANES,), c, jnp.int32)], vals)

            out_copies.append(
                pltpu.async_copy(
                    buf, out_hbm.at[pl.ds(base + j * _CHUNK, _CHUNK)],
                    osem0 if j % 2 == 0 else osem1))
        for oc in out_copies[-2:]:
            oc.wait()

    return k(idx, Wwide)


# speedup vs baseline: 1.7161x; 1.7161x over previous
import jax
import jax.numpy as jnp
from jax import lax
from jax.experimental import pallas as pl
from jax.experimental.pallas import tpu as pltpu
from jax.experimental.pallas import tpu_sc as plsc

_NC = 2
_NS = 16
_NW = _NC * _NS


def kernel(idx, W):
    B = idx.shape[0]
    NZ = W.shape[1]
    b_per_w = B // _NW

    mesh = plsc.VectorSubcoreMesh(core_axis_name="c", subcore_axis_name="s")

    @pl.kernel(
        mesh=mesh,
        out_type=jax.ShapeDtypeStruct((B, NZ), W.dtype),
        compiler_params=pltpu.CompilerParams(needs_layout_passes=False),
        scratch_types=[
            pltpu.VMEM((b_per_w,), jnp.int32),
            pltpu.VMEM((b_per_w, NZ), jnp.float32),
            pltpu.SemaphoreType.DMA,
        ],
    )
    def k(idx_hbm, table_hbm, out_hbm, idx_s, rows_v, gsem):
        wid = lax.axis_index("s") * _NC + lax.axis_index("c")
        base = wid * b_per_w
        pltpu.sync_copy(idx_hbm.at[pl.ds(base, b_per_w)], idx_s)

        @pl.loop(0, b_per_w, step=16)
        def _(i):
            vec = idx_s[pl.ds(i, 16)]
            for l in range(16):
                pltpu.async_copy(
                    table_hbm.at[pl.ds(vec[l], 1)],
                    rows_v.at[pl.ds(i + l, 1)], gsem)

        @pl.loop(0, b_per_w)
        def _(r):
            pltpu.make_async_copy(
                table_hbm.at[pl.ds(0, 1)], rows_v.at[pl.ds(r, 1)], gsem
            ).wait()

        pltpu.sync_copy(rows_v, out_hbm.at[pl.ds(base, b_per_w)])

    return k(idx, W)
